# baseline (speedup 1.0000x reference)
import functools

import jax
import jax.numpy as jnp
from jax.experimental import pallas as pl
from jax.experimental.pallas import tpu as pltpu

H = 1024
E = 16
I = 512
IS = 2048
N_GROUP = 4
GROUP_SIZE = E // N_GROUP
TOPK_GROUP = 2
TOP_K = 8
ROUTED_SCALE = 2.5

T = 2048


W128 = 128
TPR = W128 // E
BR = T * E // W128


def _roll128(v, k):
    k = k % W128
    if k == 0:
        return v
    return jnp.concatenate([v[:, k:], v[:, :k]], axis=1)


def _combine_from_scores(s, bias):
    lane = jax.lax.broadcasted_iota(jnp.int32, (BR, W128), 1)
    sub = lane % E

    def seg_rot(v, k):
        stay = (sub + k) < E
        return jnp.where(stay, _roll128(v, k), _roll128(v, k - E))

    sc = s + bias

    p = jnp.where(sub % 2 == 0, seg_rot(sc, 1), seg_rot(sc, E - 1))
    hi = jnp.maximum(sc, p)
    lo = jnp.minimum(sc, p)
    sw2 = lambda v: jnp.where(sub % 4 < 2, seg_rot(v, 2), seg_rot(v, E - 2))
    hi_o = sw2(hi)
    lo_o = sw2(lo)
    top2 = jnp.maximum(jnp.maximum(hi + hi_o, hi + lo), hi_o + lo_o)

    g_idx = sub // GROUP_SIZE
    grank = jnp.zeros((BR, W128), jnp.int32)
    for k in range(1, N_GROUP):
        gk = seg_rot(top2, GROUP_SIZE * k)
        gk_idx = (g_idx + k) % N_GROUP
        beats = jnp.logical_or(
            gk > top2, jnp.logical_and(gk == top2, gk_idx < g_idx))
        grank = grank + jnp.where(beats, 1, 0)
    gsel = grank < TOPK_GROUP

    scores_for_choice = jnp.where(gsel, sc, 0.0)

    v = scores_for_choice
    erank = jnp.zeros((BR, W128), jnp.int32)
    for k in range(1, E):
        vk = seg_rot(v, k)
        jk = (sub + k) % E
        beats = jnp.logical_or(
            vk > v, jnp.logical_and(vk == v, jk < sub))
        erank = erank + jnp.where(beats, 1, 0)
    sel = erank < TOP_K

    tw = jnp.where(sel, s, 0.0)
    denom = tw
    for k in (8, 4, 2, 1):
        denom = denom + seg_rot(denom, k)
    denom = denom + 1e-20
    return tw * (ROUTED_SCALE / denom)


def _router_kernel(s_ref, bias_ref, cmb_ref):
    cmb_ref[...] = _combine_from_scores(s_ref[...], bias_ref[...])


def _moe_kernel(cmb_ref, x_ref, wu_ref, wd_ref, wus_ref, wds_ref,
                out_ref, xbf_ref):
    i = pl.program_id(0)
    routed = i < E // 2

    @pl.when(i == 0)
    def _prologue():
        xbf_ref[...] = x_ref[...].astype(jnp.bfloat16)

    CH = T // 4
    NPAIR = E // 2


    def pair_body(init):
        wu0 = wu_ref[0].astype(jnp.bfloat16)
        wu1 = wu_ref[1].astype(jnp.bfloat16)
        wdc = jnp.concatenate([wd_ref[0], wd_ref[1]], axis=0).astype(jnp.bfloat16)
        w0 = cmb_ref[0, :, 0:1]
        w1 = cmb_ref[0, :, 1:2]
        for hf in range(4):
            r0 = hf * CH
            xh = xbf_ref[r0:r0 + CH, :]
            h0 = jnp.dot(xh, wu0, preferred_element_type=jnp.float32)
            g0 = (jnp.square(jnp.maximum(h0, 0.0)) * w0[r0:r0 + CH, :]
                  ).astype(jnp.bfloat16)
            h1 = jnp.dot(xh, wu1, preferred_element_type=jnp.float32)
            g1 = (jnp.square(jnp.maximum(h1, 0.0)) * w1[r0:r0 + CH, :]
                  ).astype(jnp.bfloat16)
            g = jnp.concatenate([g0, g1], axis=1)
            y = jnp.dot(g, wdc, preferred_element_type=jnp.float32)
            if init:
                out_ref[r0:r0 + CH, :] = y
            else:
                out_ref[r0:r0 + CH, :] += y

    @pl.when(i == 0)
    def _pair0():
        pair_body(True)

    @pl.when(jnp.logical_and(i > 0, routed))
    def _pair():
        pair_body(False)

    @pl.when(jnp.logical_not(routed))
    def _shared():
        wus = wus_ref[...].astype(jnp.bfloat16)
        wds = wds_ref[...].astype(jnp.bfloat16)
        for hf in range(4):
            r0 = hf * CH
            xh = xbf_ref[r0:r0 + CH, :]
            h = jnp.dot(xh, wus, preferred_element_type=jnp.float32)
            g = jnp.square(jnp.maximum(h, 0.0)).astype(jnp.bfloat16)
            out_ref[r0:r0 + CH, :] += jnp.dot(
                g, wds, preferred_element_type=jnp.float32)


def _build(interpret=False):
    npair = E // 2
    nsteps = npair + IS // I
    moe = pl.pallas_call(
        _moe_kernel,
        grid=(nsteps,),
        in_specs=[
            pl.BlockSpec((1, T, 2),
                         lambda i: (jnp.where(i < npair, i, npair - 1), 0, 0)),
            pl.BlockSpec((T, H), lambda i: (0, 0)),
            pl.BlockSpec((2, H, I),
                         lambda i: (jnp.where(i < npair, i, npair - 1), 0, 0)),
            pl.BlockSpec((2, I, H),
                         lambda i: (jnp.where(i < npair, i, npair - 1), 0, 0)),
            pl.BlockSpec((H, I), lambda i: (0, jnp.where(i < npair, 0, i - npair))),
            pl.BlockSpec((I, H), lambda i: (jnp.where(i < npair, 0, i - npair), 0)),
        ],
        out_specs=pl.BlockSpec((T, H), lambda i: (0, 0)),
        out_shape=jax.ShapeDtypeStruct((T, H), jnp.float32),
        scratch_shapes=[pltpu.VMEM((T, H), jnp.bfloat16)],
        compiler_params=pltpu.CompilerParams(
            dimension_semantics=("arbitrary",),
        ),
        interpret=interpret,
    )
    router = pl.pallas_call(
        _router_kernel,
        grid=(1,),
        in_specs=[
            pl.BlockSpec((BR, W128), lambda i: (0, 0)),
            pl.BlockSpec((1, W128), lambda i: (0, 0)),
        ],
        out_specs=pl.BlockSpec((BR, W128), lambda i: (0, 0)),
        out_shape=jax.ShapeDtypeStruct((BR, W128), jnp.float32),
        interpret=interpret,
    )
    return router, moe


@functools.partial(jax.jit, static_argnames=("interpret",))
def _run(hidden_states, Wg, e_bias, Wu, Wd, Wu_s, Wd_s, interpret=False):
    router, moe = _build(interpret)
    x = hidden_states.reshape(T, H)
    s = jax.nn.sigmoid(x.astype(jnp.float32) @ Wg.T)
    s128 = s.reshape(BR, W128)
    bias128 = jnp.tile(e_bias, TPR).reshape(1, W128)
    cmb = router(s128, bias128).reshape(T, E)
    cmb3 = cmb.reshape(T, E // 2, 2).transpose(1, 0, 2)
    out = moe(cmb3, x, Wu, Wd, Wu_s, Wd_s)
    return out.reshape(hidden_states.shape)


def kernel(hidden_states, Wg, e_bias, Wu, Wd, Wu_s, Wd_s):
    return _run(hidden_states, Wg, e_bias, Wu, Wd, Wu_s, Wd_s)


# speedup vs baseline: 1.1422x; 1.1422x over previous
import functools

import jax
import jax.numpy as jnp
from jax.experimental import pallas as pl
from jax.experimental.pallas import tpu as pltpu

H = 1024
E = 16
I = 512
IS = 2048
N_GROUP = 4
GROUP_SIZE = E // N_GROUP
TOPK_GROUP = 2
TOP_K = 8
ROUTED_SCALE = 2.5

T = 2048


W128 = 128
TPR = W128 // E
BR = T * E // W128


def _roll128(v, k):
    k = k % W128
    if k == 0:
        return v
    return jnp.concatenate([v[:, k:], v[:, :k]], axis=1)


def _combine_from_scores(s, bias):
    lane = jax.lax.broadcasted_iota(jnp.int32, (BR, W128), 1)
    sub = lane % E

    def seg_rot(v, k):
        stay = (sub + k) < E
        return jnp.where(stay, _roll128(v, k), _roll128(v, k - E))

    sc = s + bias

    p = jnp.where(sub % 2 == 0, seg_rot(sc, 1), seg_rot(sc, E - 1))
    hi = jnp.maximum(sc, p)
    lo = jnp.minimum(sc, p)
    sw2 = lambda v: jnp.where(sub % 4 < 2, seg_rot(v, 2), seg_rot(v, E - 2))
    hi_o = sw2(hi)
    lo_o = sw2(lo)
    top2 = jnp.maximum(jnp.maximum(hi + hi_o, hi + lo), hi_o + lo_o)

    g_idx = sub // GROUP_SIZE
    grank = jnp.zeros((BR, W128), jnp.int32)
    for k in range(1, N_GROUP):
        gk = seg_rot(top2, GROUP_SIZE * k)
        gk_idx = (g_idx + k) % N_GROUP
        beats = jnp.logical_or(
            gk > top2, jnp.logical_and(gk == top2, gk_idx < g_idx))
        grank = grank + jnp.where(beats, 1, 0)
    gsel = grank < TOPK_GROUP

    scores_for_choice = jnp.where(gsel, sc, 0.0)

    v = scores_for_choice
    erank = jnp.zeros((BR, W128), jnp.int32)
    for k in range(1, E):
        vk = seg_rot(v, k)
        jk = (sub + k) % E
        beats = jnp.logical_or(
            vk > v, jnp.logical_and(vk == v, jk < sub))
        erank = erank + jnp.where(beats, 1, 0)
    sel = erank < TOP_K

    tw = jnp.where(sel, s, 0.0)
    denom = tw
    for k in (8, 4, 2, 1):
        denom = denom + seg_rot(denom, k)
    denom = denom + 1e-20
    return tw * (ROUTED_SCALE / denom)


def _router_kernel(s_ref, bias_ref, cmb_ref):
    cmb_ref[...] = _combine_from_scores(s_ref[...], bias_ref[...])


def _moe_kernel(cmb_ref, x_ref, wu_ref, wd_ref, wus_ref, wds_ref,
                out_ref, xbf_ref):
    i = pl.program_id(0)
    routed = i < E // 2

    @pl.when(i == 0)
    def _prologue():
        xbf_ref[...] = x_ref[...].astype(jnp.bfloat16)

    CH = T // 4
    NPAIR = E // 2


    def routed_w(col):
        lane = jax.lax.broadcasted_iota(jnp.int32, (T, E), 1)
        return jnp.sum(jnp.where(lane == col, cmb_ref[...], 0.0),
                       axis=1, keepdims=True)

    def pair_body(init):
        wu0 = wu_ref[0].astype(jnp.bfloat16)
        wu1 = wu_ref[1].astype(jnp.bfloat16)
        wdc = jnp.concatenate([wd_ref[0], wd_ref[1]], axis=0).astype(jnp.bfloat16)
        w0 = routed_w(2 * i)
        w1 = routed_w(2 * i + 1)
        for hf in range(4):
            r0 = hf * CH
            xh = xbf_ref[r0:r0 + CH, :]
            h0 = jnp.dot(xh, wu0, preferred_element_type=jnp.float32)
            g0 = (jnp.square(jnp.maximum(h0, 0.0)) * w0[r0:r0 + CH, :]
                  ).astype(jnp.bfloat16)
            h1 = jnp.dot(xh, wu1, preferred_element_type=jnp.float32)
            g1 = (jnp.square(jnp.maximum(h1, 0.0)) * w1[r0:r0 + CH, :]
                  ).astype(jnp.bfloat16)
            g = jnp.concatenate([g0, g1], axis=1)
            y = jnp.dot(g, wdc, preferred_element_type=jnp.float32)
            if init:
                out_ref[r0:r0 + CH, :] = y
            else:
                out_ref[r0:r0 + CH, :] += y

    @pl.when(i == 0)
    def _pair0():
        pair_body(True)

    @pl.when(jnp.logical_and(i > 0, routed))
    def _pair():
        pair_body(False)

    @pl.when(jnp.logical_not(routed))
    def _shared():
        wus = wus_ref[...].astype(jnp.bfloat16)
        wds = wds_ref[...].astype(jnp.bfloat16)
        for hf in range(4):
            r0 = hf * CH
            xh = xbf_ref[r0:r0 + CH, :]
            h = jnp.dot(xh, wus, preferred_element_type=jnp.float32)
            g = jnp.square(jnp.maximum(h, 0.0)).astype(jnp.bfloat16)
            out_ref[r0:r0 + CH, :] += jnp.dot(
                g, wds, preferred_element_type=jnp.float32)


def _build(interpret=False):
    npair = E // 2
    nsteps = npair + IS // I
    moe = pl.pallas_call(
        _moe_kernel,
        grid=(nsteps,),
        in_specs=[
            pl.BlockSpec((T, E), lambda i: (0, 0)),
            pl.BlockSpec((T, H), lambda i: (0, 0)),
            pl.BlockSpec((2, H, I),
                         lambda i: (jnp.where(i < npair, i, npair - 1), 0, 0)),
            pl.BlockSpec((2, I, H),
                         lambda i: (jnp.where(i < npair, i, npair - 1), 0, 0)),
            pl.BlockSpec((H, I), lambda i: (0, jnp.where(i < npair, 0, i - npair))),
            pl.BlockSpec((I, H), lambda i: (jnp.where(i < npair, 0, i - npair), 0)),
        ],
        out_specs=pl.BlockSpec((T, H), lambda i: (0, 0)),
        out_shape=jax.ShapeDtypeStruct((T, H), jnp.float32),
        scratch_shapes=[pltpu.VMEM((T, H), jnp.bfloat16)],
        compiler_params=pltpu.CompilerParams(
            dimension_semantics=("arbitrary",),
        ),
        interpret=interpret,
    )
    router = pl.pallas_call(
        _router_kernel,
        grid=(1,),
        in_specs=[
            pl.BlockSpec((BR, W128), lambda i: (0, 0)),
            pl.BlockSpec((1, W128), lambda i: (0, 0)),
        ],
        out_specs=pl.BlockSpec((BR, W128), lambda i: (0, 0)),
        out_shape=jax.ShapeDtypeStruct((BR, W128), jnp.float32),
        interpret=interpret,
    )
    return router, moe


@functools.partial(jax.jit, static_argnames=("interpret",))
def _run(hidden_states, Wg, e_bias, Wu, Wd, Wu_s, Wd_s, interpret=False):
    router, moe = _build(interpret)
    x = hidden_states.reshape(T, H)
    s = jax.nn.sigmoid(x.astype(jnp.float32) @ Wg.T)
    s128 = s.reshape(BR, W128)
    bias128 = jnp.tile(e_bias, TPR).reshape(1, W128)
    cmb = router(s128, bias128).reshape(T, E)
    out = moe(cmb, x, Wu, Wd, Wu_s, Wd_s)
    return out.reshape(hidden_states.shape)


def kernel(hidden_states, Wg, e_bias, Wu, Wd, Wu_s, Wd_s):
    return _run(hidden_states, Wg, e_bias, Wu, Wd, Wu_s, Wd_s)


# speedup vs baseline: 1.1424x; 1.0002x over previous
import functools

import jax
import jax.numpy as jnp
from jax.experimental import pallas as pl
from jax.experimental.pallas import tpu as pltpu

H = 1024
E = 16
I = 512
IS = 2048
N_GROUP = 4
GROUP_SIZE = E // N_GROUP
TOPK_GROUP = 2
TOP_K = 8
ROUTED_SCALE = 2.5

T = 2048


W128 = 128
TPR = W128 // E
BR = T * E // W128


def _roll128(v, k):
    k = k % W128
    if k == 0:
        return v
    return jnp.concatenate([v[:, k:], v[:, :k]], axis=1)


def _combine_from_scores(s, bias):
    lane = jax.lax.broadcasted_iota(jnp.int32, (BR, W128), 1)
    sub = lane % E

    def seg_rot(v, k):
        stay = (sub + k) < E
        return jnp.where(stay, _roll128(v, k), _roll128(v, k - E))

    sc = s + bias

    p = jnp.where(sub % 2 == 0, seg_rot(sc, 1), seg_rot(sc, E - 1))
    hi = jnp.maximum(sc, p)
    lo = jnp.minimum(sc, p)
    sw2 = lambda v: jnp.where(sub % 4 < 2, seg_rot(v, 2), seg_rot(v, E - 2))
    hi_o = sw2(hi)
    lo_o = sw2(lo)
    top2 = jnp.maximum(jnp.maximum(hi + hi_o, hi + lo), hi_o + lo_o)

    g_idx = sub // GROUP_SIZE
    grank = jnp.zeros((BR, W128), jnp.int32)
    for k in range(1, N_GROUP):
        gk = seg_rot(top2, GROUP_SIZE * k)
        gk_idx = (g_idx + k) % N_GROUP
        beats = jnp.logical_or(
            gk > top2, jnp.logical_and(gk == top2, gk_idx < g_idx))
        grank = grank + jnp.where(beats, 1, 0)
    gsel = grank < TOPK_GROUP

    scores_for_choice = jnp.where(gsel, sc, 0.0)

    v = scores_for_choice
    erank = jnp.zeros((BR, W128), jnp.int32)
    for k in range(1, E):
        vk = seg_rot(v, k)
        jk = (sub + k) % E
        beats = jnp.logical_or(
            vk > v, jnp.logical_and(vk == v, jk < sub))
        erank = erank + jnp.where(beats, 1, 0)
    sel = erank < TOP_K

    tw = jnp.where(sel, s, 0.0)
    denom = tw
    for k in (8, 4, 2, 1):
        denom = denom + seg_rot(denom, k)
    denom = denom + 1e-20
    return tw * (ROUTED_SCALE / denom)


def _router_kernel(s_ref, bias_ref, cmb_ref):
    cmb_ref[...] = _combine_from_scores(s_ref[...], bias_ref[...])


def _moe_kernel(cmb_ref, x_ref, wu_ref, wd_ref, wus_ref, wds_ref,
                out_ref, xbf_ref):
    i = pl.program_id(0)
    routed = i < E // 2

    @pl.when(i == 0)
    def _prologue():
        xbf_ref[...] = x_ref[...].astype(jnp.bfloat16)

    CH = T // 4

    def routed_w(col):
        lane = jax.lax.broadcasted_iota(jnp.int32, (T, E), 1)
        return jnp.sum(jnp.where(lane == col, cmb_ref[...], 0.0),
                       axis=1, keepdims=True)

    def pair_body(init):
        wu0 = wu_ref[0].astype(jnp.bfloat16)
        wu1 = wu_ref[1].astype(jnp.bfloat16)
        wdc = jnp.concatenate([wd_ref[0], wd_ref[1]], axis=0).astype(jnp.bfloat16)
        w0 = routed_w(2 * i)
        w1 = routed_w(2 * i + 1)
        for hf in range(4):
            r0 = hf * CH
            xh = xbf_ref[r0:r0 + CH, :]
            h0 = jnp.dot(xh, wu0, preferred_element_type=jnp.float32)
            g0 = (jnp.square(jnp.maximum(h0, 0.0)) * w0[r0:r0 + CH, :]
                  ).astype(jnp.bfloat16)
            h1 = jnp.dot(xh, wu1, preferred_element_type=jnp.float32)
            g1 = (jnp.square(jnp.maximum(h1, 0.0)) * w1[r0:r0 + CH, :]
                  ).astype(jnp.bfloat16)
            g = jnp.concatenate([g0, g1], axis=1)
            y = jnp.dot(g, wdc, preferred_element_type=jnp.float32)
            if init:
                out_ref[r0:r0 + CH, :] = y
            else:
                out_ref[r0:r0 + CH, :] += y

    @pl.when(i == 0)
    def _pair0():
        pair_body(True)

    @pl.when(jnp.logical_and(i > 0, routed))
    def _pair():
        pair_body(False)

    @pl.when(jnp.logical_not(routed))
    def _shared():
        wus = wus_ref[...].astype(jnp.bfloat16)
        wds = wds_ref[...].astype(jnp.bfloat16)
        for hf in range(4):
            r0 = hf * CH
            xh = xbf_ref[r0:r0 + CH, :]
            h = jnp.dot(xh, wus, preferred_element_type=jnp.float32)
            g = jnp.square(jnp.maximum(h, 0.0)).astype(jnp.bfloat16)
            out_ref[r0:r0 + CH, :] += jnp.dot(
                g, wds, preferred_element_type=jnp.float32)


def _build(interpret=False):
    npair = E // 2
    nsteps = npair + IS // I
    moe = pl.pallas_call(
        _moe_kernel,
        grid=(nsteps,),
        in_specs=[
            pl.BlockSpec((T, E), lambda i: (0, 0)),
            pl.BlockSpec((T, H), lambda i: (0, 0)),
            pl.BlockSpec((2, H, I),
                         lambda i: (jnp.where(i < npair, i, npair - 1), 0, 0)),
            pl.BlockSpec((2, I, H),
                         lambda i: (jnp.where(i < npair, i, npair - 1), 0, 0)),
            pl.BlockSpec((H, I), lambda i: (0, jnp.where(i < npair, 0, i - npair))),
            pl.BlockSpec((I, H), lambda i: (jnp.where(i < npair, 0, i - npair), 0)),
        ],
        out_specs=pl.BlockSpec((T, H), lambda i: (0, 0)),
        out_shape=jax.ShapeDtypeStruct((T, H), jnp.float32),
        scratch_shapes=[pltpu.VMEM((T, H), jnp.bfloat16)],
        compiler_params=pltpu.CompilerParams(
            dimension_semantics=("arbitrary",),
        ),
        interpret=interpret,
    )
    router = pl.pallas_call(
        _router_kernel,
        grid=(1,),
        in_specs=[
            pl.BlockSpec((BR, W128), lambda i: (0, 0)),
            pl.BlockSpec((1, W128), lambda i: (0, 0)),
        ],
        out_specs=pl.BlockSpec((BR, W128), lambda i: (0, 0)),
        out_shape=jax.ShapeDtypeStruct((BR, W128), jnp.float32),
        interpret=interpret,
    )
    return router, moe


@functools.partial(jax.jit, static_argnames=("interpret",))
def _run(hidden_states, Wg, e_bias, Wu, Wd, Wu_s, Wd_s, interpret=False):
    router, moe = _build(interpret)
    x = hidden_states.reshape(T, H)
    s = jax.nn.sigmoid(x.astype(jnp.float32) @ Wg.T)
    s128 = s.reshape(BR, W128)
    bias128 = jnp.tile(e_bias, TPR).reshape(1, W128)
    cmb = router(s128, bias128).reshape(T, E)
    out = moe(cmb, x, Wu, Wd, Wu_s, Wd_s)
    return out.reshape(hidden_states.shape)


def kernel(hidden_states, Wg, e_bias, Wu, Wd, Wu_s, Wd_s):
    return _run(hidden_states, Wg, e_bias, Wu, Wd, Wu_s, Wd_s)
